# baseline (speedup 1.0000x reference)
import functools

import jax
import jax.numpy as jnp
from jax import lax
from jax.experimental import pallas as pl
from jax.experimental.pallas import tpu as pltpu
from jax.experimental.pallas import tpu_sc as plsc

N = 10000
E = 320000
D = 128
NP = 10240
NC = 2
NS = 16
NW = NC * NS
EPW = E // NW
B = 96
NCH = 105
EPWA = NCH * B
APAD = NW * EPWA - E
RPS = NP // NS
RB = 512
GRID = NP // RB

_mesh = plsc.VectorSubcoreMesh(
    core_axis_name="c", subcore_axis_name="s", num_cores=NC, num_subcores=NS
)


BD = 128
NCHD = 79
EPWD = NCHD * BD
EPAD = NW * EPWD - E


@functools.partial(
    pl.kernel,
    out_type=jax.ShapeDtypeStruct((NC, NP), jnp.float32),
    mesh=_mesh,
    scratch_types=[
        pltpu.VMEM((NCHD, BD), jnp.int32),
        pltpu.VMEM((BD,), jnp.float32),
        pltpu.VMEM((RPS,), jnp.float32),
        pltpu.VMEM_SHARED((NP,), jnp.float32),
        pltpu.SemaphoreType.DMA,
        pltpu.SemaphoreType.DMA,
    ],
)
def _sc_degree(dst_hbm, deg_hbm, didx, ones, zstrip, acc, sem0, sem1):
    cid = lax.axis_index("c")
    sid = lax.axis_index("s")
    w = cid * NS + sid

    pltpu.async_copy(dst_hbm.at[w], didx, sem0)

    def fill_ones(i, _):
        ones[pl.ds(i * 16, 16)] = jnp.ones((16,), jnp.float32)
        return 0

    lax.fori_loop(0, BD // 16, fill_ones, 0)

    def fill_zero(i, _):
        zstrip[pl.ds(i * 16, 16)] = jnp.zeros((16,), jnp.float32)
        return 0

    lax.fori_loop(0, RPS // 16, fill_zero, 0)

    pltpu.sync_copy(zstrip, acc.at[pl.ds(sid * RPS, RPS)])
    pltpu.make_async_copy(dst_hbm.at[w], didx, sem0).wait()
    plsc.subcore_barrier()

    pltpu.async_copy(ones, acc.at[didx.at[0]], sem0, add=True)

    def body(i, _):
        c0 = 2 * i
        c1 = 2 * i + 1
        c2 = 2 * i + 2
        pltpu.async_copy(ones, acc.at[didx.at[c1]], sem1, add=True)
        pltpu.make_async_copy(ones, acc.at[didx.at[c0]], sem0).wait()
        pltpu.async_copy(ones, acc.at[didx.at[c2]], sem0, add=True)
        pltpu.make_async_copy(ones, acc.at[didx.at[c1]], sem1).wait()
        return 0

    lax.fori_loop(0, (NCHD - 1) // 2, body, 0)
    pltpu.make_async_copy(ones, acc.at[didx.at[NCHD - 1]], sem0).wait()
    plsc.subcore_barrier()
    pltpu.sync_copy(
        acc.at[pl.ds(sid * RPS, RPS)], deg_hbm.at[cid, pl.ds(sid * RPS, RPS)]
    )


@functools.partial(
    pl.kernel,
    out_type=jax.ShapeDtypeStruct((NC, NP, D), jnp.float32),
    mesh=_mesh,
    scratch_types=[
        pltpu.VMEM((EPWA,), jnp.int32),
        pltpu.VMEM((NCH, B), jnp.int32),
        pltpu.VMEM((B, D), jnp.float32),
        pltpu.VMEM((B, D), jnp.float32),
        pltpu.VMEM_SHARED((NP, D), jnp.float32),
        pltpu.SemaphoreType.DMA,
        pltpu.SemaphoreType.DMA,
    ],
)
def _sc_aggregate(
    h_hbm, src_hbm, dst_hbm, out_hbm, sidx, didx, rows0, rows1, acc, sem0, sem1
):
    cid = lax.axis_index("c")
    sid = lax.axis_index("s")
    w = cid * NS + sid
    base = sid * RPS

    pltpu.async_copy(src_hbm.at[w], sidx, sem0)
    pltpu.async_copy(dst_hbm.at[w], didx, sem1)

    def fill_zero(r, _):
        for k in range(D // 16):
            rows0[r, pl.ds(k * 16, 16)] = jnp.zeros((16,), jnp.float32)
        return 0

    lax.fori_loop(0, B, fill_zero, 0)
    pltpu.make_async_copy(src_hbm.at[w], sidx, sem0).wait()
    pltpu.make_async_copy(dst_hbm.at[w], didx, sem1).wait()

    pltpu.async_copy(h_hbm.at[sidx.at[pl.ds(0, B)]], rows1, sem1)

    def zero_acc(j, _):
        pltpu.sync_copy(rows0, acc.at[pl.ds(base + j * B, B)])
        return 0

    lax.fori_loop(0, RPS // B, zero_acc, 0)
    ZREM = RPS - (RPS // B) * B
    pltpu.sync_copy(
        rows0.at[pl.ds(0, ZREM)], acc.at[pl.ds(base + (RPS // B) * B, ZREM)]
    )
    plsc.subcore_barrier()

    def body(i, _):
        c0 = 2 * i
        c1 = 2 * i + 1
        c2 = 2 * i + 2
        pltpu.async_copy(h_hbm.at[sidx.at[pl.ds(c1 * B, B)]], rows0, sem0)
        pltpu.make_async_copy(h_hbm.at[sidx.at[pl.ds(c0 * B, B)]], rows1, sem1).wait()
        pltpu.sync_copy(rows1, acc.at[didx.at[c0]], add=True)
        pltpu.async_copy(h_hbm.at[sidx.at[pl.ds(c2 * B, B)]], rows1, sem1)
        pltpu.make_async_copy(h_hbm.at[sidx.at[pl.ds(c1 * B, B)]], rows0, sem0).wait()
        pltpu.sync_copy(rows0, acc.at[didx.at[c1]], add=True)
        return 0

    lax.fori_loop(0, (NCH - 1) // 2, body, 0)
    pltpu.make_async_copy(h_hbm.at[sidx.at[pl.ds((NCH - 1) * B, B)]], rows1, sem1).wait()
    pltpu.sync_copy(rows1, acc.at[didx.at[NCH - 1]], add=True)
    plsc.subcore_barrier()
    pltpu.sync_copy(
        acc.at[pl.ds(base, RPS)], out_hbm.at[cid, pl.ds(base, RPS)]
    )


def _dinv(degp_ref):
    deg = degp_ref[0, :] + degp_ref[1, :] + 1.0
    return lax.rsqrt(deg)


def _tc_pre_body(x_ref, w_ref, degp_ref, o_ref):
    dinv = _dinv(degp_ref)
    o_ref[...] = (
        jnp.dot(x_ref[...], w_ref[...], preferred_element_type=jnp.float32)
        * dinv[:, None]
    )


def _tc_mid_body(pp_ref, hh_ref, degp_ref, b_ref, w_ref, o_ref):
    dinv = _dinv(degp_ref)
    p = pp_ref[0] + pp_ref[1] + hh_ref[...]
    h = jnp.maximum(p * dinv[:, None] + b_ref[...][None, :], 0.0)
    o_ref[...] = (
        jnp.dot(h, w_ref[...], preferred_element_type=jnp.float32)
        * dinv[:, None]
    )


def _tc_post_body(pp_ref, hh_ref, degp_ref, b_ref, o_ref):
    dinv = _dinv(degp_ref)
    p = pp_ref[0] + pp_ref[1] + hh_ref[...]
    h = jnp.maximum(p * dinv[:, None] + b_ref[...][None, :], 0.0)
    m = jnp.max(h, axis=1, keepdims=True)
    e = jnp.exp(h - m)
    o_ref[...] = (h - m) - jnp.log(jnp.sum(e, axis=1, keepdims=True))


_row_spec = pl.BlockSpec((RB, D), lambda i: (i, 0))
_w_spec = pl.BlockSpec((D, D), lambda i: (0, 0))
_degp_spec = pl.BlockSpec((NC, RB), lambda i: (0, i))
_pp_spec = pl.BlockSpec((NC, RB, D), lambda i: (0, i, 0))
_b_spec = pl.BlockSpec((D,), lambda i: (0,))

_tc_pre = pl.pallas_call(
    _tc_pre_body,
    grid=(GRID,),
    in_specs=[_row_spec, _w_spec, _degp_spec],
    out_specs=_row_spec,
    out_shape=jax.ShapeDtypeStruct((NP, D), jnp.float32),
)

_tc_mid = pl.pallas_call(
    _tc_mid_body,
    grid=(GRID,),
    in_specs=[_pp_spec, _row_spec, _degp_spec, _b_spec, _w_spec],
    out_specs=_row_spec,
    out_shape=jax.ShapeDtypeStruct((NP, D), jnp.float32),
)

_tc_post = pl.pallas_call(
    _tc_post_body,
    grid=(GRID,),
    in_specs=[_pp_spec, _row_spec, _degp_spec, _b_spec],
    out_specs=_row_spec,
    out_shape=jax.ShapeDtypeStruct((NP, D), jnp.float32),
)


@jax.jit
def _gcn(x, edge_index, W0, b0, W1, b1, W2, b2):
    pad = jnp.full((APAD,), N, jnp.int32)
    src = jnp.concatenate([edge_index[0], pad]).reshape(NW, EPWA)
    dst = jnp.concatenate([edge_index[1], pad]).reshape(NW, NCH, B)
    dst_deg = jnp.concatenate(
        [edge_index[1], jnp.full((EPAD,), N, jnp.int32)]
    ).reshape(NW, NCHD, BD)
    x_p = jnp.pad(x, ((0, NP - N), (0, 0)))

    degp = _sc_degree(dst_deg)
    hh0 = _tc_pre(x_p, W0, degp)
    pp0 = _sc_aggregate(hh0, src, dst)
    hh1 = _tc_mid(pp0, hh0, degp, b0, W1)
    pp1 = _sc_aggregate(hh1, src, dst)
    hh2 = _tc_mid(pp1, hh1, degp, b1, W2)
    pp2 = _sc_aggregate(hh2, src, dst)
    out = _tc_post(pp2, hh2, degp, b2)
    return out[:N]


def kernel(x, edge_index, W0, b0, W1, b1, W2, b2):
    return _gcn(x, edge_index, W0, b0, W1, b1, W2, b2)


# speedup vs baseline: 1.6547x; 1.6547x over previous
import functools

import jax
import jax.numpy as jnp
from jax import lax
from jax.experimental import pallas as pl
from jax.experimental.pallas import tpu as pltpu
from jax.experimental.pallas import tpu_sc as plsc

N = 10000
E = 320000
D = 128
NP = 10240
NC = 2
NS = 16
NW = NC * NS
EPW = E // NW
B = 96
NCH = 105
EPWA = NCH * B
APAD = NW * EPWA - E
RPS = NP // NS
RB = 512
GRID = NP // RB

_mesh = plsc.VectorSubcoreMesh(
    core_axis_name="c", subcore_axis_name="s", num_cores=NC, num_subcores=NS
)


BD = 128
NCHD = 79
EPWD = NCHD * BD
EPAD = NW * EPWD - E


@functools.partial(
    pl.kernel,
    out_type=jax.ShapeDtypeStruct((NC, NP), jnp.float32),
    mesh=_mesh,
    scratch_types=[
        pltpu.VMEM((NCHD, BD), jnp.int32),
        pltpu.VMEM((BD,), jnp.float32),
        pltpu.VMEM((RPS,), jnp.float32),
        pltpu.VMEM_SHARED((NP,), jnp.float32),
        pltpu.SemaphoreType.DMA,
        pltpu.SemaphoreType.DMA,
    ],
)
def _sc_degree(dst_hbm, deg_hbm, didx, ones, zstrip, acc, sem0, sem1):
    cid = lax.axis_index("c")
    sid = lax.axis_index("s")
    w = cid * NS + sid

    pltpu.async_copy(dst_hbm.at[w], didx, sem0)

    def fill_ones(i, _):
        ones[pl.ds(i * 16, 16)] = jnp.ones((16,), jnp.float32)
        return 0

    lax.fori_loop(0, BD // 16, fill_ones, 0)

    def fill_zero(i, _):
        zstrip[pl.ds(i * 16, 16)] = jnp.zeros((16,), jnp.float32)
        return 0

    lax.fori_loop(0, RPS // 16, fill_zero, 0)

    pltpu.sync_copy(zstrip, acc.at[pl.ds(sid * RPS, RPS)])
    pltpu.make_async_copy(dst_hbm.at[w], didx, sem0).wait()
    plsc.subcore_barrier()

    pltpu.async_copy(ones, acc.at[didx.at[0]], sem0, add=True)

    def body(i, _):
        c0 = 2 * i
        c1 = 2 * i + 1
        c2 = 2 * i + 2
        pltpu.async_copy(ones, acc.at[didx.at[c1]], sem1, add=True)
        pltpu.make_async_copy(ones, acc.at[didx.at[c0]], sem0).wait()
        pltpu.async_copy(ones, acc.at[didx.at[c2]], sem0, add=True)
        pltpu.make_async_copy(ones, acc.at[didx.at[c1]], sem1).wait()
        return 0

    lax.fori_loop(0, (NCHD - 1) // 2, body, 0)
    pltpu.make_async_copy(ones, acc.at[didx.at[NCHD - 1]], sem0).wait()
    plsc.subcore_barrier()
    pltpu.sync_copy(
        acc.at[pl.ds(sid * RPS, RPS)], deg_hbm.at[cid, pl.ds(sid * RPS, RPS)]
    )


@functools.partial(
    pl.kernel,
    out_type=jax.ShapeDtypeStruct((NC, NP, D), jnp.float32),
    mesh=_mesh,
    scratch_types=[
        pltpu.VMEM((EPWA,), jnp.int32),
        pltpu.VMEM((NCH, B), jnp.int32),
        pltpu.VMEM((B, D), jnp.float32),
        pltpu.VMEM((B, D), jnp.float32),
        pltpu.VMEM_SHARED((NP, D), jnp.float32),
        pltpu.SemaphoreType.DMA,
        pltpu.SemaphoreType.DMA,
    ],
)
def _sc_aggregate(
    h_hbm, src_hbm, dst_hbm, out_hbm, sidx, didx, rows0, rows1, acc, sem0, sem1
):
    cid = lax.axis_index("c")
    sid = lax.axis_index("s")
    w = cid * NS + sid
    base = sid * RPS

    pltpu.async_copy(src_hbm.at[w], sidx, sem0)
    pltpu.async_copy(dst_hbm.at[w], didx, sem1)

    def fill_zero(r, _):
        for k in range(D // 16):
            rows0[r, pl.ds(k * 16, 16)] = jnp.zeros((16,), jnp.float32)
        return 0

    lax.fori_loop(0, B, fill_zero, 0)
    pltpu.make_async_copy(src_hbm.at[w], sidx, sem0).wait()
    pltpu.make_async_copy(dst_hbm.at[w], didx, sem1).wait()

    pltpu.async_copy(h_hbm.at[sidx.at[pl.ds(0, B)]], rows1, sem1)

    def zero_acc(j, _):
        pltpu.sync_copy(rows0, acc.at[pl.ds(base + j * B, B)])
        return 0

    lax.fori_loop(0, RPS // B, zero_acc, 0)
    ZREM = RPS - (RPS // B) * B
    pltpu.sync_copy(
        rows0.at[pl.ds(0, ZREM)], acc.at[pl.ds(base + (RPS // B) * B, ZREM)]
    )
    plsc.subcore_barrier()

    def body(i, _):
        c0 = 2 * i
        c1 = 2 * i + 1
        c2 = 2 * i + 2
        pltpu.async_copy(h_hbm.at[sidx.at[pl.ds(c1 * B, B)]], rows0, sem0)
        pltpu.make_async_copy(h_hbm.at[sidx.at[pl.ds(c0 * B, B)]], rows1, sem1).wait()
        pltpu.sync_copy(rows1, acc.at[didx.at[c0]], add=True)
        pltpu.async_copy(h_hbm.at[sidx.at[pl.ds(c2 * B, B)]], rows1, sem1)
        pltpu.make_async_copy(h_hbm.at[sidx.at[pl.ds(c1 * B, B)]], rows0, sem0).wait()
        pltpu.sync_copy(rows0, acc.at[didx.at[c1]], add=True)
        return 0

    lax.fori_loop(0, (NCH - 1) // 2, body, 0)
    pltpu.make_async_copy(h_hbm.at[sidx.at[pl.ds((NCH - 1) * B, B)]], rows1, sem1).wait()
    pltpu.sync_copy(rows1, acc.at[didx.at[NCH - 1]], add=True)
    plsc.subcore_barrier()
    pltpu.sync_copy(
        acc.at[pl.ds(base, RPS)], out_hbm.at[cid, pl.ds(base, RPS)]
    )


def _dinv(degp_ref):
    deg = degp_ref[0, :] + degp_ref[1, :] + 1.0
    return lax.rsqrt(deg)


def _tc_pre_body(x_ref, w_ref, degp_ref, o_ref):
    dinv = _dinv(degp_ref)
    o_ref[...] = (
        jnp.dot(x_ref[...], w_ref[...], preferred_element_type=jnp.float32)
        * dinv[:, None]
    )


def _tc_mid_body(pp_ref, hh_ref, degp_ref, b_ref, w_ref, o_ref):
    dinv = _dinv(degp_ref)
    p = pp_ref[0] + pp_ref[1] + hh_ref[...]
    h = jnp.maximum(p * dinv[:, None] + b_ref[...][None, :], 0.0)
    o_ref[...] = (
        jnp.dot(h, w_ref[...], preferred_element_type=jnp.float32)
        * dinv[:, None]
    )


def _tc_post_body(pp_ref, hh_ref, degp_ref, b_ref, o_ref):
    dinv = _dinv(degp_ref)
    p = pp_ref[0] + pp_ref[1] + hh_ref[...]
    h = jnp.maximum(p * dinv[:, None] + b_ref[...][None, :], 0.0)
    m = jnp.max(h, axis=1, keepdims=True)
    e = jnp.exp(h - m)
    o_ref[...] = (h - m) - jnp.log(jnp.sum(e, axis=1, keepdims=True))


_row_spec = pl.BlockSpec((RB, D), lambda i: (i, 0))
_w_spec = pl.BlockSpec((D, D), lambda i: (0, 0))
_degp_spec = pl.BlockSpec((NC, RB), lambda i: (0, i))
_pp_spec = pl.BlockSpec((NC, RB, D), lambda i: (0, i, 0))
_b_spec = pl.BlockSpec((D,), lambda i: (0,))

_tc_pre = pl.pallas_call(
    _tc_pre_body,
    grid=(GRID,),
    in_specs=[_row_spec, _w_spec, _degp_spec],
    out_specs=_row_spec,
    out_shape=jax.ShapeDtypeStruct((NP, D), jnp.float32),
)

_tc_mid = pl.pallas_call(
    _tc_mid_body,
    grid=(GRID,),
    in_specs=[_pp_spec, _row_spec, _degp_spec, _b_spec, _w_spec],
    out_specs=_row_spec,
    out_shape=jax.ShapeDtypeStruct((NP, D), jnp.float32),
)

_tc_post = pl.pallas_call(
    _tc_post_body,
    grid=(GRID,),
    in_specs=[_pp_spec, _row_spec, _degp_spec, _b_spec],
    out_specs=_row_spec,
    out_shape=jax.ShapeDtypeStruct((NP, D), jnp.float32),
)


@jax.jit
def _gcn(x, edge_index, W0, b0, W1, b1, W2, b2):
    pad = N + jnp.arange(APAD, dtype=jnp.int32) % (NP - N)
    src = jnp.concatenate([edge_index[0], pad]).reshape(NW, EPWA)
    dst = jnp.concatenate([edge_index[1], pad]).reshape(NW, NCH, B)
    dst_deg = jnp.concatenate(
        [edge_index[1], N + jnp.arange(EPAD, dtype=jnp.int32) % (NP - N)]
    ).reshape(NW, NCHD, BD)
    x_p = jnp.pad(x, ((0, NP - N), (0, 0)))

    degp = _sc_degree(dst_deg)
    hh0 = _tc_pre(x_p, W0, degp)
    pp0 = _sc_aggregate(hh0, src, dst)
    hh1 = _tc_mid(pp0, hh0, degp, b0, W1)
    pp1 = _sc_aggregate(hh1, src, dst)
    hh2 = _tc_mid(pp1, hh1, degp, b1, W2)
    pp2 = _sc_aggregate(hh2, src, dst)
    out = _tc_post(pp2, hh2, degp, b2)
    return out[:N]


def kernel(x, edge_index, W0, b0, W1, b1, W2, b2):
    return _gcn(x, edge_index, W0, b0, W1, b1, W2, b2)
